# Initial kernel scaffold; baseline (speedup 1.0000x reference)
import jax
import jax.numpy as jnp
from jax.experimental import pallas as pl


def kernel(embeddings, buckets, delta):
    raise NotImplementedError("write your pallas kernel here")


# speedup vs baseline: 1.7011x; 1.7011x over previous
import functools

import jax
import jax.numpy as jnp
from jax import lax
from jax.experimental import pallas as pl
from jax.experimental.pallas import tpu as pltpu
from jax.experimental.pallas import tpu_sc as plsc

N_BUCKETS = 65536
EMBED_DIM = 128
BATCH = 16384

N_CORES = 2
N_SUBCORES = 16
LANES = 16

N_RANGES = 8
RANGE = N_BUCKETS // N_RANGES
PASSES = N_RANGES // N_CORES
CHUNK = BATCH // N_SUBCORES
N_VREGS = CHUNK // LANES


def _body(emb, buckets, delta, out, idx_v, blist, plist, ebuf, dbuf, acc):
    c = lax.axis_index("c")
    s = lax.axis_index("s")
    iota = lax.iota(jnp.int32, LANES)

    pltpu.sync_copy(buckets.at[pl.ds(s * CHUNK, CHUNK)], idx_v)

    for p in range(PASSES):
        r = 2 * p + c

        def scan_body(g, off_vec):
            b16 = idx_v[pl.ds(g * LANES, LANES)]
            m = (b16 >> 13) == r
            mi = jnp.where(m, 1, 0).astype(jnp.int32)
            lanepos = off_vec + plsc.cumsum(mi) - 1
            pos16 = s * CHUNK + g * LANES + iota
            plsc.store_scatter(blist, [lanepos], b16, mask=m)
            plsc.store_scatter(plist, [lanepos], pos16, mask=m)
            return off_vec + plsc.all_reduce_population_count(m)

        off_vec = lax.fori_loop(
            0, N_VREGS, scan_body, jnp.zeros((LANES,), jnp.int32))
        count = jnp.max(off_vec)
        n_chunks = (count + LANES - 1) // LANES

        def p1(k, carry):
            valid = (k * LANES + iota) < count
            b16 = blist[pl.ds(k * LANES, LANES)]
            bg = jnp.where(valid, b16, iota)
            pltpu.sync_copy(emb.at[bg], ebuf)
            locb = jnp.where(valid, b16 & (RANGE - 1), RANGE + iota)
            pltpu.sync_copy(ebuf, acc.at[locb])
            return carry

        lax.fori_loop(0, n_chunks, p1, 0)
        plsc.subcore_barrier()

        def p2(k, carry):
            valid = (k * LANES + iota) < count
            b16 = blist[pl.ds(k * LANES, LANES)]
            p16 = plist[pl.ds(k * LANES, LANES)]
            pg = jnp.where(valid, p16, iota)
            pltpu.sync_copy(delta.at[pg], dbuf)
            locb = jnp.where(valid, b16 & (RANGE - 1), RANGE + iota)
            pltpu.sync_copy(dbuf, acc.at[locb], add=True)
            return carry

        lax.fori_loop(0, n_chunks, p2, 0)
        plsc.subcore_barrier()

        def p3(k, carry):
            valid = (k * LANES + iota) < count
            b16 = blist[pl.ds(k * LANES, LANES)]
            p16 = plist[pl.ds(k * LANES, LANES)]
            locb = jnp.where(valid, b16 & (RANGE - 1), RANGE + iota)
            pltpu.sync_copy(acc.at[locb], ebuf)
            pw = jnp.where(valid, p16, BATCH + iota)
            pltpu.sync_copy(ebuf, out.at[pw])
            return carry

        lax.fori_loop(0, n_chunks, p3, 0)
        plsc.subcore_barrier()


_engram = pl.kernel(
    _body,
    out_type=jax.ShapeDtypeStruct((BATCH + LANES, EMBED_DIM), jnp.float32),
    mesh=plsc.VectorSubcoreMesh(core_axis_name="c", subcore_axis_name="s"),
    compiler_params=pltpu.CompilerParams(needs_layout_passes=False),
    scratch_types=[
        pltpu.VMEM((CHUNK,), jnp.int32),
        pltpu.VMEM((CHUNK,), jnp.int32),
        pltpu.VMEM((CHUNK,), jnp.int32),
        pltpu.VMEM((LANES, EMBED_DIM), jnp.float32),
        pltpu.VMEM((LANES, EMBED_DIM), jnp.float32),
        pltpu.VMEM_SHARED((RANGE + LANES, EMBED_DIM), jnp.float32),
    ],
)


def kernel(embeddings, buckets, delta):
    return _engram(embeddings, buckets, delta)[:BATCH]


# speedup vs baseline: 2.3246x; 1.3665x over previous
import jax
import jax.numpy as jnp
from jax import lax
from jax.experimental import pallas as pl
from jax.experimental.pallas import tpu as pltpu
from jax.experimental.pallas import tpu_sc as plsc

N_BUCKETS = 65536
EMBED_DIM = 128
BATCH = 16384

N_SUBCORES = 16
LANES = 16

N_RANGES = 8
RANGE = N_BUCKETS // N_RANGES
PASSES = N_RANGES // 2
CHUNK = BATCH // N_SUBCORES
N_VREGS = CHUNK // LANES
C = 128
MAXCH = CHUNK // C


def _body(emb, buckets, delta, out, idx_v, bg, locb, pd, pw, ebuf, dbuf, acc):
    c = lax.axis_index("c")
    s = lax.axis_index("s")
    iota = lax.iota(jnp.int32, LANES)

    pltpu.sync_copy(buckets.at[pl.ds(s * CHUNK, CHUNK)], idx_v)

    for p in range(PASSES):
        r = 2 * p + c

        def scan_body(g, off_vec):
            b16 = idx_v[pl.ds(g * LANES, LANES)]
            m = (b16 >> 13) == r
            mi = jnp.where(m, 1, 0).astype(jnp.int32)
            lanepos = off_vec + plsc.cumsum(mi) - 1
            row, col = lanepos >> 7, lanepos & (C - 1)
            pos16 = s * CHUNK + g * LANES + iota
            plsc.store_scatter(bg, [row, col], b16, mask=m)
            plsc.store_scatter(locb, [row, col], b16 & (RANGE - 1), mask=m)
            plsc.store_scatter(pd, [row, col], pos16, mask=m)
            plsc.store_scatter(pw, [row, col], pos16, mask=m)
            return off_vec + plsc.all_reduce_population_count(m)

        off_vec = lax.fori_loop(
            0, N_VREGS, scan_body, jnp.zeros((LANES,), jnp.int32))
        count = jnp.max(off_vec)
        n_chunks = (count + C - 1) // C

        def pad_body(v, carry):
            idx = v * LANES + iota
            m = idx >= count
            row, col = idx >> 7, idx & (C - 1)
            plsc.store_scatter(bg, [row, col], idx & (N_BUCKETS - 1), mask=m)
            plsc.store_scatter(locb, [row, col], RANGE + col, mask=m)
            plsc.store_scatter(pd, [row, col], idx & (BATCH - 1), mask=m)
            plsc.store_scatter(pw, [row, col], BATCH + col, mask=m)
            return carry

        lax.fori_loop(count // LANES, n_chunks * (C // LANES), pad_body, 0)

        def p1(k, carry):
            pltpu.sync_copy(emb.at[bg.at[k]], ebuf)
            pltpu.sync_copy(ebuf, acc.at[locb.at[k]])
            return carry

        lax.fori_loop(0, n_chunks, p1, 0)
        plsc.subcore_barrier()

        def p2(k, carry):
            pltpu.sync_copy(delta.at[pd.at[k]], dbuf)
            pltpu.sync_copy(dbuf, acc.at[locb.at[k]], add=True)
            return carry

        lax.fori_loop(0, n_chunks, p2, 0)
        plsc.subcore_barrier()

        def p3(k, carry):
            pltpu.sync_copy(acc.at[locb.at[k]], ebuf)
            pltpu.sync_copy(ebuf, out.at[pw.at[k]])
            return carry

        lax.fori_loop(0, n_chunks, p3, 0)
        plsc.subcore_barrier()


_engram = pl.kernel(
    _body,
    out_type=jax.ShapeDtypeStruct((BATCH + C, EMBED_DIM), jnp.float32),
    mesh=plsc.VectorSubcoreMesh(core_axis_name="c", subcore_axis_name="s"),
    compiler_params=pltpu.CompilerParams(needs_layout_passes=False),
    scratch_types=[
        pltpu.VMEM((CHUNK,), jnp.int32),
        pltpu.VMEM((MAXCH, C), jnp.int32),
        pltpu.VMEM((MAXCH, C), jnp.int32),
        pltpu.VMEM((MAXCH, C), jnp.int32),
        pltpu.VMEM((MAXCH, C), jnp.int32),
        pltpu.VMEM((C, EMBED_DIM), jnp.float32),
        pltpu.VMEM((C, EMBED_DIM), jnp.float32),
        pltpu.VMEM_SHARED((RANGE + C, EMBED_DIM), jnp.float32),
    ],
)


def kernel(embeddings, buckets, delta):
    return _engram(embeddings, buckets, delta)[:BATCH]


# speedup vs baseline: 2.4030x; 1.0338x over previous
import jax
import jax.numpy as jnp
from jax import lax
from jax.experimental import pallas as pl
from jax.experimental.pallas import tpu as pltpu
from jax.experimental.pallas import tpu_sc as plsc

N_BUCKETS = 65536
EMBED_DIM = 128
BATCH = 16384

N_SUBCORES = 16
LANES = 16

N_RANGES = 8
RANGE = N_BUCKETS // N_RANGES
PASSES = N_RANGES // 2
CHUNK = BATCH // N_SUBCORES
N_VREGS = CHUNK // LANES
C = 128
MAXCH = CHUNK // C


def _body(emb, buckets, delta, out,
          idx_v, bg, locbr, pd, locb2, pwr, ebuf, dbuf, acc):
    c = lax.axis_index("c")
    s = lax.axis_index("s")
    iota = lax.iota(jnp.int32, LANES)

    pltpu.sync_copy(buckets.at[pl.ds(s * CHUNK, CHUNK)], idx_v)

    for p in range(PASSES):
        r = 2 * p + c

        with jax.named_scope("scan"):
            def scan_body(g, off_vec):
                b16 = idx_v[pl.ds(g * LANES, LANES)]
                m = (b16 >> 13) == r
                mi = jnp.where(m, 1, 0).astype(jnp.int32)
                lanepos = off_vec + plsc.cumsum(mi) - 1
                row, col = lanepos >> 7, lanepos & (C - 1)
                pos16 = s * CHUNK + g * LANES + iota
                lb = b16 & (RANGE - 1)
                plsc.store_scatter(bg, [row, col], b16, mask=m)
                plsc.store_scatter(locbr, [row, col], lb, mask=m)
                plsc.store_scatter(pd, [row, col], pos16, mask=m)
                plsc.store_scatter(locb2, [row, col], lb, mask=m)
                plsc.store_scatter(pwr, [row, col], pos16, mask=m)
                return off_vec + plsc.all_reduce_population_count(m)

            off_vec = lax.fori_loop(
                0, N_VREGS, scan_body, jnp.zeros((LANES,), jnp.int32))
            count = jnp.max(off_vec)
            n_chunks = (count + C - 1) // C

            def pad_body(v, carry):
                idx = v * LANES + iota
                m = idx >= count
                row, col = idx >> 7, idx & (C - 1)
                j = idx % count
                jrow, jcol = j >> 7, j & (C - 1)
                plsc.store_scatter(
                    bg, [row, col], plsc.load_gather(bg, [jrow, jcol]), mask=m)
                plsc.store_scatter(
                    locbr, [row, col],
                    plsc.load_gather(locbr, [jrow, jcol]), mask=m)
                plsc.store_scatter(
                    pwr, [row, col],
                    plsc.load_gather(pwr, [jrow, jcol]), mask=m)
                plsc.store_scatter(pd, [row, col], idx & (BATCH - 1), mask=m)
                plsc.store_scatter(locb2, [row, col], RANGE + col, mask=m)
                return carry

            lax.fori_loop(count // LANES, n_chunks * (C // LANES), pad_body, 0)

        with jax.named_scope("p1"):
            def p1(k, carry):
                pltpu.sync_copy(emb.at[bg.at[k]], ebuf)
                pltpu.sync_copy(ebuf, acc.at[locbr.at[k]])
                return carry

            lax.fori_loop(0, n_chunks, p1, 0)
            plsc.subcore_barrier()

        with jax.named_scope("p2"):
            def p2(k, carry):
                pltpu.sync_copy(delta.at[pd.at[k]], dbuf)
                pltpu.sync_copy(dbuf, acc.at[locb2.at[k]], add=True)
                return carry

            lax.fori_loop(0, n_chunks, p2, 0)
            plsc.subcore_barrier()

        with jax.named_scope("p3"):
            def p3(k, carry):
                pltpu.sync_copy(acc.at[locbr.at[k]], ebuf)
                pltpu.sync_copy(ebuf, out.at[pwr.at[k]])
                return carry

            lax.fori_loop(0, n_chunks, p3, 0)
            plsc.subcore_barrier()


_engram = pl.kernel(
    _body,
    out_type=jax.ShapeDtypeStruct((BATCH, EMBED_DIM), jnp.float32),
    mesh=plsc.VectorSubcoreMesh(core_axis_name="c", subcore_axis_name="s"),
    compiler_params=pltpu.CompilerParams(needs_layout_passes=False),
    scratch_types=[
        pltpu.VMEM((CHUNK,), jnp.int32),
        pltpu.VMEM((MAXCH, C), jnp.int32),
        pltpu.VMEM((MAXCH, C), jnp.int32),
        pltpu.VMEM((MAXCH, C), jnp.int32),
        pltpu.VMEM((MAXCH, C), jnp.int32),
        pltpu.VMEM((MAXCH, C), jnp.int32),
        pltpu.VMEM((C, EMBED_DIM), jnp.float32),
        pltpu.VMEM((C, EMBED_DIM), jnp.float32),
        pltpu.VMEM_SHARED((RANGE + C, EMBED_DIM), jnp.float32),
    ],
)


def kernel(embeddings, buckets, delta):
    return _engram(embeddings, buckets, delta)


# speedup vs baseline: 3.3068x; 1.3761x over previous
import jax
import jax.numpy as jnp
from jax import lax
from jax.experimental import pallas as pl
from jax.experimental.pallas import tpu as pltpu
from jax.experimental.pallas import tpu_sc as plsc

N_BUCKETS = 65536
EMBED_DIM = 128
BATCH = 16384

N_SUBCORES = 16
LANES = 16

N_RANGES = 8
RANGE = N_BUCKETS // N_RANGES
PASSES = N_RANGES // 2
CHUNK = BATCH // N_SUBCORES
N_VREGS = CHUNK // LANES
C = 32
NCH = CHUNK // C
UNROLL = 6


def _body(emb, buckets, delta, out,
          idx_v, locbr, pwr, locb2, eb, db, ob,
          se0, se1, sd0, sd1, so0, so1, acc):
    se = [se0, se1]
    sd = [sd0, sd1]
    so = [so0, so1]
    c = lax.axis_index("c")
    s = lax.axis_index("s")
    iota = lax.iota(jnp.int32, LANES)

    pltpu.sync_copy(buckets.at[pl.ds(s * CHUNK, CHUNK)], idx_v)

    with jax.named_scope("scan"):
        def scan_body(g, offs):
            b16 = idx_v[pl.ds(g * LANES, LANES)]
            pos16 = s * CHUNK + g * LANES + iota
            lb = b16 & (RANGE - 1)
            rng = b16 >> 13
            new_offs = []
            for p in range(PASSES):
                m = rng == 2 * p + c
                mi = jnp.where(m, 1, 0).astype(jnp.int32)
                lanepos = offs[p] + plsc.cumsum(mi) - 1
                row = p * NCH + (lanepos >> 5)
                col = lanepos & (C - 1)
                plsc.store_scatter(locbr, [row, col], lb, mask=m)
                plsc.store_scatter(pwr, [row, col], pos16, mask=m)
                new_offs.append(offs[p] + plsc.all_reduce_population_count(m))
            return tuple(new_offs)

        offs = lax.fori_loop(
            0, N_VREGS, scan_body,
            tuple(jnp.zeros((LANES,), jnp.int32) for _ in range(PASSES)))
        counts = [jnp.max(offs[p]) for p in range(PASSES)]
        nchs = [(counts[p] + C - 1) // C for p in range(PASSES)]

        for p in range(PASSES):
            count = counts[p]
            n = nchs[p]

            def pad_body(v, carry, p=p, count=count, n=n):
                idx = v * LANES + iota
                m = idx >= count
                row = p * NCH + (idx >> 5)
                col = idx & (C - 1)
                j = jnp.where(m, idx % jnp.maximum(count, 1), idx)
                jrow = p * NCH + (j >> 5)
                jcol = j & (C - 1)
                lbj = plsc.load_gather(locbr, [jrow, jcol])
                plsc.store_scatter(locbr, [row, col], lbj, mask=m)
                plsc.store_scatter(
                    pwr, [row, col],
                    plsc.load_gather(pwr, [jrow, jcol]), mask=m)
                tcol = idx - (n - 1) * C
                plsc.store_scatter(
                    locb2, [jnp.full((LANES,), p, jnp.int32), tcol],
                    jnp.where(m, RANGE + tcol, lbj))
                return carry

            lax.fori_loop(
                (n - 1) * (C // LANES), n * (C // LANES), pad_body, 0)

    def d_emb(p, k, slot):
        roff = (2 * p + c) * RANGE
        return pltpu.make_async_copy(
            emb.at[pl.ds(roff, RANGE)].at[locbr.at[p * NCH + k]],
            eb.at[slot], se[slot])

    def d_delta(p, k, slot):
        return pltpu.make_async_copy(
            delta.at[pwr.at[p * NCH + k]], db.at[slot], sd[slot])

    def d_acc(p, k, slot):
        return pltpu.make_async_copy(
            acc.at[locbr.at[p * NCH + k]], ob.at[slot], so[slot])

    def fire(desc, p, k):
        @pl.when(k < nchs[p])
        def _():
            desc.start()

    for k in range(2):
        fire(d_emb(0, k, k), 0, k)
        fire(d_delta(0, k, k), 0, k)

    for p in range(PASSES):
        n = nchs[p]

        with jax.named_scope("p1"):
            for k in range(UNROLL):
                @pl.when(k < n)
                def _(p=p, k=k):
                    d_emb(p, k, k % 2).wait()
                    pltpu.sync_copy(eb.at[k % 2], acc.at[locbr.at[p * NCH + k]])
                if k + 2 < UNROLL:
                    fire(d_emb(p, k + 2, k % 2), p, k + 2)

            def p1_fb(k, carry, p=p):
                roff = (2 * p + c) * RANGE
                pltpu.sync_copy(
                    emb.at[pl.ds(roff, RANGE)].at[locbr.at[p * NCH + k]],
                    eb.at[0])
                pltpu.sync_copy(eb.at[0], acc.at[locbr.at[p * NCH + k]])
                return carry

            lax.fori_loop(UNROLL, n, p1_fb, 0)
            plsc.subcore_barrier()

        with jax.named_scope("p2"):
            for k in range(UNROLL):
                @pl.when(k < n)
                def _(p=p, k=k):
                    d_delta(p, k, k % 2).wait()

                @pl.when(k < n - 1)
                def _(p=p, k=k):
                    pltpu.sync_copy(
                        db.at[k % 2], acc.at[locbr.at[p * NCH + k]], add=True)

                @pl.when(k == n - 1)
                def _(p=p, k=k):
                    pltpu.sync_copy(
                        db.at[k % 2], acc.at[locb2.at[p]],
                        add=True)

                if k + 2 < UNROLL:
                    fire(d_delta(p, k + 2, k % 2), p, k + 2)

            def p2_fb(k, carry, p=p, n=n):
                pltpu.sync_copy(delta.at[pwr.at[p * NCH + k]], db.at[0])

                @pl.when(k < n - 1)
                def _():
                    pltpu.sync_copy(
                        db.at[0], acc.at[locbr.at[p * NCH + k]], add=True)

                @pl.when(k == n - 1)
                def _():
                    pltpu.sync_copy(
                        db.at[0], acc.at[locb2.at[p]], add=True)

                return carry

            lax.fori_loop(UNROLL, n, p2_fb, 0)
            plsc.subcore_barrier()

        with jax.named_scope("p3"):
            for k in range(2):
                fire(d_acc(p, k, k), p, k)
            if p + 1 < PASSES:
                for k in range(2):
                    fire(d_emb(p + 1, k, k), p + 1, k)
                    fire(d_delta(p + 1, k, k), p + 1, k)
            for k in range(UNROLL):
                @pl.when(k < n)
                def _(p=p, k=k):
                    d_acc(p, k, k % 2).wait()
                    pltpu.sync_copy(ob.at[k % 2], out.at[pwr.at[p * NCH + k]])
                if k + 2 < UNROLL:
                    fire(d_acc(p, k + 2, k % 2), p, k + 2)

            def p3_fb(k, carry, p=p):
                pltpu.sync_copy(acc.at[locbr.at[p * NCH + k]], ob.at[0])
                pltpu.sync_copy(ob.at[0], out.at[pwr.at[p * NCH + k]])
                return carry

            lax.fori_loop(UNROLL, n, p3_fb, 0)
            plsc.subcore_barrier()


_engram = pl.kernel(
    _body,
    out_type=jax.ShapeDtypeStruct((BATCH, EMBED_DIM), jnp.float32),
    mesh=plsc.VectorSubcoreMesh(core_axis_name="c", subcore_axis_name="s"),
    compiler_params=pltpu.CompilerParams(needs_layout_passes=False),
    scratch_types=[
        pltpu.VMEM((CHUNK,), jnp.int32),
        pltpu.VMEM((PASSES * NCH, C), jnp.int32),
        pltpu.VMEM((PASSES * NCH, C), jnp.int32),
        pltpu.VMEM((PASSES, C), jnp.int32),
        pltpu.VMEM((2, C, EMBED_DIM), jnp.float32),
        pltpu.VMEM((2, C, EMBED_DIM), jnp.float32),
        pltpu.VMEM((2, C, EMBED_DIM), jnp.float32),
        pltpu.SemaphoreType.DMA,
        pltpu.SemaphoreType.DMA,
        pltpu.SemaphoreType.DMA,
        pltpu.SemaphoreType.DMA,
        pltpu.SemaphoreType.DMA,
        pltpu.SemaphoreType.DMA,
        pltpu.VMEM_SHARED((RANGE + C, EMBED_DIM), jnp.float32),
    ],
)


def kernel(embeddings, buckets, delta):
    return _engram(embeddings, buckets, delta)


# speedup vs baseline: 3.4256x; 1.0359x over previous
import jax
import jax.numpy as jnp
from jax import lax
from jax.experimental import pallas as pl
from jax.experimental.pallas import tpu as pltpu
from jax.experimental.pallas import tpu_sc as plsc

N_BUCKETS = 65536
EMBED_DIM = 128
BATCH = 16384

N_SUBCORES = 16
LANES = 16

N_RANGES = 8
RANGE = N_BUCKETS // N_RANGES
PASSES = N_RANGES // 2
CHUNK = BATCH // N_SUBCORES
N_VREGS = CHUNK // LANES
C = 32
NCH = CHUNK // C
UNROLL = 6
NSLOT = 3


def _body(emb, buckets, delta, out,
          idx_v, locbr, pwr, locb2, eb, db, ob,
          se0, se1, se2, sd0, sd1, sd2, so0, so1, so2,
          ss0, ss1, ss2, acc):
    se = [se0, se1, se2]
    sd = [sd0, sd1, sd2]
    so = [so0, so1, so2]
    ss = [ss0, ss1, ss2]
    c = lax.axis_index("c")
    s = lax.axis_index("s")
    iota = lax.iota(jnp.int32, LANES)

    pltpu.sync_copy(buckets.at[pl.ds(s * CHUNK, CHUNK)], idx_v)

    with jax.named_scope("scan"):
        def scan_body(g, offs):
            b16 = idx_v[pl.ds(g * LANES, LANES)]
            pos16 = s * CHUNK + g * LANES + iota
            lb = b16 & (RANGE - 1)
            rng = b16 >> 13
            new_offs = []
            for p in range(PASSES):
                m = rng == 2 * p + c
                mi = jnp.where(m, 1, 0).astype(jnp.int32)
                lanepos = offs[p] + plsc.cumsum(mi) - 1
                row = p * NCH + (lanepos >> 5)
                col = lanepos & (C - 1)
                plsc.store_scatter(locbr, [row, col], lb, mask=m)
                plsc.store_scatter(pwr, [row, col], pos16, mask=m)
                new_offs.append(offs[p] + plsc.all_reduce_population_count(m))
            return tuple(new_offs)

        offs = lax.fori_loop(
            0, N_VREGS, scan_body,
            tuple(jnp.zeros((LANES,), jnp.int32) for _ in range(PASSES)))
        counts = [jnp.max(offs[p]) for p in range(PASSES)]
        nchs = [(counts[p] + C - 1) // C for p in range(PASSES)]

    def d_emb(p, k, slot):
        roff = (2 * p + c) * RANGE
        return pltpu.make_async_copy(
            emb.at[pl.ds(roff, RANGE)].at[locbr.at[p * NCH + k]],
            eb.at[slot], se[slot])

    def d_delta(p, k, slot):
        return pltpu.make_async_copy(
            delta.at[pwr.at[p * NCH + k]], db.at[slot], sd[slot])

    def d_acc(p, k, slot):
        return pltpu.make_async_copy(
            acc.at[locbr.at[p * NCH + k]], ob.at[slot], so[slot])

    def s_emb(p, k, slot):
        return pltpu.make_async_copy(
            eb.at[slot], acc.at[locbr.at[p * NCH + k]], ss[slot])

    def s_out(p, k, slot):
        return pltpu.make_async_copy(
            ob.at[slot], out.at[pwr.at[p * NCH + k]], ss[slot])

    def fire(desc, p, k):
        @pl.when(k < nchs[p])
        def _():
            desc.start()

    def pad_fill(p):
        count = counts[p]
        n = nchs[p]

        def pad_body(v, carry, p=p, count=count, n=n):
            idx = v * LANES + iota
            m = idx >= count
            row = p * NCH + (idx >> 5)
            col = idx & (C - 1)
            j = jnp.where(m, idx % jnp.maximum(count, 1), idx)
            jrow = p * NCH + (j >> 5)
            jcol = j & (C - 1)
            lbj = plsc.load_gather(locbr, [jrow, jcol])
            plsc.store_scatter(locbr, [row, col], lbj, mask=m)
            plsc.store_scatter(
                pwr, [row, col],
                plsc.load_gather(pwr, [jrow, jcol]), mask=m)
            tcol = idx - (n - 1) * C
            plsc.store_scatter(
                locb2, [jnp.full((LANES,), p, jnp.int32), tcol],
                jnp.where(m, RANGE + tcol, lbj))
            return carry

        lax.fori_loop(
            (n - 1) * (C // LANES), n * (C // LANES), pad_body, 0)

    pad_fill(0)
    for k in range(2):
        fire(d_emb(0, k, k), 0, k)
        fire(d_delta(0, k, k), 0, k)
    for p in range(1, PASSES):
        pad_fill(p)

    def run_phase(p, n, d_g, s_start, s_drain, fallback, prefire_here,
                  nslot):
        if prefire_here:
            for k in range(2):
                fire(d_g(p, k, k), p, k)
        for k in range(UNROLL):
            @pl.when(k < n)
            def _(p=p, k=k):
                d_g(p, k, k % nslot).wait()
                s_start(p, k, k % nslot)
            if k + 2 < UNROLL:
                @pl.when(k + 2 < n)
                def _(p=p, k=k):
                    j = k - (nslot - 2)
                    if j >= 0:
                        s_drain(p, j, j % nslot)
                    d_g(p, k + 2, (k + 2) % nslot).start()
        for j in range(UNROLL):
            if j <= UNROLL - 3 - (nslot - 2):
                cond = (j < n) & (j + nslot >= n)
            else:
                cond = j < n

            @pl.when(cond)
            def _(p=p, j=j):
                s_drain(p, j, j % nslot)

        lax.fori_loop(UNROLL, n, fallback, 0)
        plsc.subcore_barrier()

    for p in range(PASSES):
        n = nchs[p]

        with jax.named_scope("p1"):
            def p1_fb(k, carry, p=p):
                roff = (2 * p + c) * RANGE
                pltpu.sync_copy(
                    emb.at[pl.ds(roff, RANGE)].at[locbr.at[p * NCH + k]],
                    eb.at[0])
                pltpu.sync_copy(eb.at[0], acc.at[locbr.at[p * NCH + k]])
                return carry

            run_phase(
                p, n, d_emb,
                lambda p, k, sl: s_emb(p, k, sl).start(),
                lambda p, k, sl: s_emb(p, k, sl).wait(),
                p1_fb, prefire_here=False, nslot=3)

        with jax.named_scope("p2"):
            def s2_start(p, k, sl, n=n):
                @pl.when(k < n - 1)
                def _():
                    pltpu.make_async_copy(
                        db.at[sl], acc.at[locbr.at[p * NCH + k]],
                        ss[sl]).start(add=True)

                @pl.when(k == n - 1)
                def _():
                    pltpu.make_async_copy(
                        db.at[sl], acc.at[locb2.at[p]],
                        ss[sl]).start(add=True)

            def s2_drain(p, k, sl):
                pltpu.make_async_copy(
                    db.at[sl], acc.at[locbr.at[p * NCH + k]], ss[sl]).wait()

            def p2_fb(k, carry, p=p, n=n):
                pltpu.sync_copy(delta.at[pwr.at[p * NCH + k]], db.at[0])

                @pl.when(k < n - 1)
                def _():
                    pltpu.sync_copy(
                        db.at[0], acc.at[locbr.at[p * NCH + k]], add=True)

                @pl.when(k == n - 1)
                def _():
                    pltpu.sync_copy(
                        db.at[0], acc.at[locb2.at[p]], add=True)

                return carry

            run_phase(p, n, d_delta, s2_start, s2_drain, p2_fb,
                      prefire_here=False, nslot=2)

        with jax.named_scope("p3"):
            if p + 1 < PASSES:
                for k in range(2):
                    fire(d_emb(p + 1, k, k), p + 1, k)
                    fire(d_delta(p + 1, k, k), p + 1, k)

            def p3_fb(k, carry, p=p):
                pltpu.sync_copy(acc.at[locbr.at[p * NCH + k]], ob.at[0])
                pltpu.sync_copy(ob.at[0], out.at[pwr.at[p * NCH + k]])
                return carry

            run_phase(
                p, n, d_acc,
                lambda p, k, sl: s_out(p, k, sl).start(),
                lambda p, k, sl: s_out(p, k, sl).wait(),
                p3_fb, prefire_here=True, nslot=2)


_engram = pl.kernel(
    _body,
    out_type=jax.ShapeDtypeStruct((BATCH, EMBED_DIM), jnp.float32),
    mesh=plsc.VectorSubcoreMesh(core_axis_name="c", subcore_axis_name="s"),
    compiler_params=pltpu.CompilerParams(needs_layout_passes=False),
    scratch_types=[
        pltpu.VMEM((CHUNK,), jnp.int32),
        pltpu.VMEM((PASSES * NCH, C), jnp.int32),
        pltpu.VMEM((PASSES * NCH, C), jnp.int32),
        pltpu.VMEM((PASSES, C), jnp.int32),
        pltpu.VMEM((3, C, EMBED_DIM), jnp.float32),
        pltpu.VMEM((2, C, EMBED_DIM), jnp.float32),
        pltpu.VMEM((2, C, EMBED_DIM), jnp.float32),
        pltpu.SemaphoreType.DMA,
        pltpu.SemaphoreType.DMA,
        pltpu.SemaphoreType.DMA,
        pltpu.SemaphoreType.DMA,
        pltpu.SemaphoreType.DMA,
        pltpu.SemaphoreType.DMA,
        pltpu.SemaphoreType.DMA,
        pltpu.SemaphoreType.DMA,
        pltpu.SemaphoreType.DMA,
        pltpu.SemaphoreType.DMA,
        pltpu.SemaphoreType.DMA,
        pltpu.SemaphoreType.DMA,
        pltpu.VMEM_SHARED((RANGE + C, EMBED_DIM), jnp.float32),
    ],
)


def kernel(embeddings, buckets, delta):
    return _engram(embeddings, buckets, delta)


# speedup vs baseline: 3.4307x; 1.0015x over previous
import jax
import jax.numpy as jnp
from jax import lax
from jax.experimental import pallas as pl
from jax.experimental.pallas import tpu as pltpu
from jax.experimental.pallas import tpu_sc as plsc

N_BUCKETS = 65536
EMBED_DIM = 128
BATCH = 16384

N_SUBCORES = 16
LANES = 16

N_RANGES = 8
RANGE = N_BUCKETS // N_RANGES
PASSES = N_RANGES // 2
CHUNK = BATCH // N_SUBCORES
N_VREGS = CHUNK // LANES
C = 32
NCH = CHUNK // C
UNROLL = 6
NSLOT = 3


def _body(emb, buckets, delta, out,
          idx_v, locbr, pwr, locb2, eb, db, ob,
          se0, se1, se2, sd0, sd1, sd2, so0, so1, so2,
          ss0, ss1, ss2, acc):
    se = [se0, se1, se2]
    sd = [sd0, sd1, sd2]
    so = [so0, so1, so2]
    ss = [ss0, ss1, ss2]
    c = lax.axis_index("c")
    s = lax.axis_index("s")
    iota = lax.iota(jnp.int32, LANES)

    pltpu.sync_copy(buckets.at[pl.ds(s * CHUNK, CHUNK)], idx_v)

    with jax.named_scope("scan"):
        def scan_body(g, offs):
            b16 = idx_v[pl.ds(g * LANES, LANES)]
            pos16 = s * CHUNK + g * LANES + iota
            lb = b16 & (RANGE - 1)
            rng = b16 >> 13
            new_offs = []
            for p in range(PASSES):
                m = rng == 2 * p + c
                mi = jnp.where(m, 1, 0).astype(jnp.int32)
                lanepos = offs[p] + plsc.cumsum(mi) - 1
                row = p * NCH + (lanepos >> 5)
                col = lanepos & (C - 1)
                plsc.store_scatter(locbr, [row, col], lb, mask=m)
                plsc.store_scatter(pwr, [row, col], pos16, mask=m)
                new_offs.append(offs[p] + plsc.all_reduce_population_count(m))
            return tuple(new_offs)

        offs = lax.fori_loop(
            0, N_VREGS, scan_body,
            tuple(jnp.zeros((LANES,), jnp.int32) for _ in range(PASSES)))
        counts = [jnp.max(offs[p]) for p in range(PASSES)]
        nchs = [(counts[p] + C - 1) // C for p in range(PASSES)]

    def d_emb(p, k, slot):
        roff = (2 * p + c) * RANGE
        return pltpu.make_async_copy(
            emb.at[pl.ds(roff, RANGE)].at[locbr.at[p * NCH + k]],
            eb.at[slot], se[slot])

    def d_delta(p, k, slot):
        return pltpu.make_async_copy(
            delta.at[pwr.at[p * NCH + k]], db.at[slot], sd[slot])

    def d_acc(p, k, slot):
        return pltpu.make_async_copy(
            acc.at[locbr.at[p * NCH + k]], ob.at[slot], so[slot])

    def s_emb(p, k, slot):
        return pltpu.make_async_copy(
            eb.at[slot], acc.at[locbr.at[p * NCH + k]], ss[slot])

    def s_out(p, k, slot):
        return pltpu.make_async_copy(
            ob.at[slot], out.at[pwr.at[p * NCH + k]], ss[slot])

    def fire(desc, p, k):
        @pl.when(k < nchs[p])
        def _():
            desc.start()

    def pad_fill(p):
        count = counts[p]
        n = nchs[p]

        def pad_body(v, carry, p=p, count=count, n=n):
            idx = v * LANES + iota
            m = idx >= count
            row = p * NCH + (idx >> 5)
            col = idx & (C - 1)
            j = jnp.where(m, idx % jnp.maximum(count, 1), idx)
            jrow = p * NCH + (j >> 5)
            jcol = j & (C - 1)
            lbj = plsc.load_gather(locbr, [jrow, jcol])
            plsc.store_scatter(locbr, [row, col], lbj, mask=m)
            plsc.store_scatter(
                pwr, [row, col],
                plsc.load_gather(pwr, [jrow, jcol]), mask=m)
            tcol = idx - (n - 1) * C
            plsc.store_scatter(
                locb2, [jnp.full((LANES,), p, jnp.int32), tcol],
                jnp.where(m, RANGE + tcol, lbj))
            return carry

        lax.fori_loop(
            (n - 1) * (C // LANES), n * (C // LANES), pad_body, 0)

    pad_fill(0)
    for k in range(2):
        fire(d_emb(0, k, k), 0, k)
        fire(d_delta(0, k, k), 0, k)
    for p in range(1, PASSES):
        pad_fill(p)

    def run_phase(p, n, d_g, s_start, s_drain, fallback, prefire_here,
                  nslot, barrier=True):
        if prefire_here:
            for k in range(2):
                fire(d_g(p, k, k), p, k)
        for k in range(UNROLL):
            @pl.when(k < n)
            def _(p=p, k=k):
                d_g(p, k, k % nslot).wait()
                s_start(p, k, k % nslot)
            if k + 2 < UNROLL:
                @pl.when(k + 2 < n)
                def _(p=p, k=k):
                    j = k - (nslot - 2)
                    if j >= 0:
                        s_drain(p, j, j % nslot)
                    d_g(p, k + 2, (k + 2) % nslot).start()
        for j in range(UNROLL):
            if j <= UNROLL - 3 - (nslot - 2):
                cond = (j < n) & (j + nslot >= n)
            else:
                cond = j < n

            @pl.when(cond)
            def _(p=p, j=j):
                s_drain(p, j, j % nslot)

        lax.fori_loop(UNROLL, n, fallback, 0)
        if barrier:
            plsc.subcore_barrier()

    for p in range(PASSES):
        n = nchs[p]

        with jax.named_scope("p1"):
            def p1_fb(k, carry, p=p):
                roff = (2 * p + c) * RANGE
                pltpu.sync_copy(
                    emb.at[pl.ds(roff, RANGE)].at[locbr.at[p * NCH + k]],
                    eb.at[0])
                pltpu.sync_copy(eb.at[0], acc.at[locbr.at[p * NCH + k]])
                return carry

            run_phase(
                p, n, d_emb,
                lambda p, k, sl: s_emb(p, k, sl).start(),
                lambda p, k, sl: s_emb(p, k, sl).wait(),
                p1_fb, prefire_here=False, nslot=3)

        with jax.named_scope("p2"):
            def s2_start(p, k, sl, n=n):
                @pl.when(k < n - 1)
                def _():
                    pltpu.make_async_copy(
                        db.at[sl], acc.at[locbr.at[p * NCH + k]],
                        ss[sl]).start(add=True)

                @pl.when(k == n - 1)
                def _():
                    pltpu.make_async_copy(
                        db.at[sl], acc.at[locb2.at[p]],
                        ss[sl]).start(add=True)

            def s2_drain(p, k, sl):
                pltpu.make_async_copy(
                    db.at[sl], acc.at[locbr.at[p * NCH + k]], ss[sl]).wait()

            def p2_fb(k, carry, p=p, n=n):
                pltpu.sync_copy(delta.at[pwr.at[p * NCH + k]], db.at[0])

                @pl.when(k < n - 1)
                def _():
                    pltpu.sync_copy(
                        db.at[0], acc.at[locbr.at[p * NCH + k]], add=True)

                @pl.when(k == n - 1)
                def _():
                    pltpu.sync_copy(
                        db.at[0], acc.at[locb2.at[p]], add=True)

                return carry

            run_phase(p, n, d_delta, s2_start, s2_drain, p2_fb,
                      prefire_here=False, nslot=2)

        with jax.named_scope("p3"):
            for k in range(2):
                fire(d_acc(p, k, k), p, k)
            if p + 1 < PASSES:
                for k in range(2):
                    fire(d_emb(p + 1, k, k), p + 1, k)
                    fire(d_delta(p + 1, k, k), p + 1, k)

            def p3_fb(k, carry, p=p):
                pltpu.sync_copy(acc.at[locbr.at[p * NCH + k]], ob.at[0])
                pltpu.sync_copy(ob.at[0], out.at[pwr.at[p * NCH + k]])
                return carry

            run_phase(
                p, n, d_acc,
                lambda p, k, sl: s_out(p, k, sl).start(),
                lambda p, k, sl: s_out(p, k, sl).wait(),
                p3_fb, prefire_here=False, nslot=2,
                barrier=(p + 1 < PASSES))


_engram = pl.kernel(
    _body,
    out_type=jax.ShapeDtypeStruct((BATCH, EMBED_DIM), jnp.float32),
    mesh=plsc.VectorSubcoreMesh(core_axis_name="c", subcore_axis_name="s"),
    compiler_params=pltpu.CompilerParams(needs_layout_passes=False),
    scratch_types=[
        pltpu.VMEM((CHUNK,), jnp.int32),
        pltpu.VMEM((PASSES * NCH, C), jnp.int32),
        pltpu.VMEM((PASSES * NCH, C), jnp.int32),
        pltpu.VMEM((PASSES, C), jnp.int32),
        pltpu.VMEM((3, C, EMBED_DIM), jnp.float32),
        pltpu.VMEM((2, C, EMBED_DIM), jnp.float32),
        pltpu.VMEM((2, C, EMBED_DIM), jnp.float32),
        pltpu.SemaphoreType.DMA,
        pltpu.SemaphoreType.DMA,
        pltpu.SemaphoreType.DMA,
        pltpu.SemaphoreType.DMA,
        pltpu.SemaphoreType.DMA,
        pltpu.SemaphoreType.DMA,
        pltpu.SemaphoreType.DMA,
        pltpu.SemaphoreType.DMA,
        pltpu.SemaphoreType.DMA,
        pltpu.SemaphoreType.DMA,
        pltpu.SemaphoreType.DMA,
        pltpu.SemaphoreType.DMA,
        pltpu.VMEM_SHARED((RANGE + C, EMBED_DIM), jnp.float32),
    ],
)


def kernel(embeddings, buckets, delta):
    return _engram(embeddings, buckets, delta)
